# baseline (speedup 1.0000x reference)
import math

import jax
import jax.numpy as jnp
from jax.experimental import pallas as pl
from jax.experimental.pallas import tpu as pltpu

Rcr = 5.2
Rca = 3.5
NS = 4
EtaR = 16.0
EtaA = 8.0
NSHFR = 16
NSHFA = 4
NSHFZ = 8
ANG_SUB = NSHFA * NSHFZ
NPAIRS = NS * (NS + 1) // 2
RAD_LEN = NS * NSHFR
ANG_LEN = NPAIRS * ANG_SUB
AEV_LEN = RAD_LEN + ANG_LEN
BM = 32

SHFR = [0.9 + 0.26875 * f for f in range(NSHFR)]
SHFA = [0.9 + 0.65 * a for a in range(NSHFA)]
SHFZ = [(2 * z + 1) * math.pi / 16.0 for z in range(NSHFZ)]

f32 = jnp.float32
i32 = jnp.int32


def _pow_zeta(x):
    x = x * x
    x = x * x
    x = x * x
    x = x * x
    return x * x


def _iota2(shape, dim):
    return jax.lax.broadcasted_iota(i32, shape, dim)


def _nt(a, b):
    return jax.lax.dot_general(a, b, (((1,), (1,)), ((), ())),
                               preferred_element_type=f32)


def _nn(a, b):
    return jax.lax.dot_general(a, b, (((1,), (0,)), ((), ())),
                               preferred_element_type=f32)


def _aev_kernel(species_ref, coords_ref, coords_t_ref, out_ref):
    A = species_ref.shape[-1]
    JK = A * A
    R = BM * A

    dxs, dys, dzs = [], [], []
    for b in range(BM):
        xc = coords_ref[b]
        xr = coords_t_ref[b]
        dxs.append(jnp.broadcast_to(xc[:, 0:1], (A, A))
                   - jnp.broadcast_to(xr[0:1, :], (A, A)))
        dys.append(jnp.broadcast_to(xc[:, 1:2], (A, A))
                   - jnp.broadcast_to(xr[1:2, :], (A, A)))
        dzs.append(jnp.broadcast_to(xc[:, 2:3], (A, A))
                   - jnp.broadcast_to(xr[2:3, :], (A, A)))
    dx = jnp.concatenate(dxs, axis=0)
    dy = jnp.concatenate(dys, axis=0)
    dz = jnp.concatenate(dzs, axis=0)
    d = jnp.sqrt(dx * dx + dy * dy + dz * dz)

    ne = jnp.where(_iota2((R, A), 0) % A == _iota2((R, A), 1),
                   0.0, 1.0).astype(f32)

    spf = jnp.concatenate([species_ref[b] for b in range(BM)],
                          axis=0).astype(f32)

    fc_r = (0.5 * jnp.cos(d * (math.pi / Rcr)) + 0.5)
    fc_r = fc_r * jnp.where(d <= Rcr, 1.0, 0.0) * ne

    rep4 = jnp.where(_iota2((BM * NS, BM), 0) // NS == _iota2((BM * NS, BM), 1),
                     1.0, 0.0).astype(f32)
    spR4 = _nn(rep4, spf)
    S_cat = jnp.where(
        jnp.abs(spR4 - (_iota2((BM * NS, A), 0) % NS).astype(f32)) < 0.5,
        1.0, 0.0).astype(f32)

    m4 = jnp.where(_iota2((R, BM * NS), 0) // A == _iota2((R, BM * NS), 1) // NS,
                   1.0, 0.0).astype(f32)

    radial = jnp.zeros((R, RAD_LEN), dtype=f32)
    q4 = _iota2((BM * NS, RAD_LEN), 1)
    s4 = _iota2((BM * NS, RAD_LEN), 0) % NS
    for f in range(NSHFR):
        dd = d - SHFR[f]
        rt = 0.25 * jnp.exp(-EtaR * dd * dd) * fc_r
        rad = _nt(rt, S_cat) * m4
        Qf = jnp.where(q4 == s4 * NSHFR + f, 1.0, 0.0).astype(f32)
        radial = radial + _nn(rad, Qf)

    T = A * (A - 1) // 2
    TP = 512
    e_row = _iota2((A, TP), 0)
    e_col = _iota2((A, TP), 1)
    offj = e_row * (A - 1) - (e_row * (e_row - 1)) // 2
    E_J = jnp.where((e_col >= offj) & (e_col < offj + (A - 1 - e_row))
                    & (e_col < T), 1.0, 0.0).astype(f32)
    t_f = _iota2((1, TP), 1).astype(f32)
    disc = jnp.maximum((2 * A - 1) * (2 * A - 1) - 8.0 * t_f, 0.0)
    j_t = jnp.floor(((2 * A - 1) - jnp.sqrt(disc)) * 0.5)
    off_t = j_t * A - j_t * (j_t + 1.0) * 0.5
    k_t = t_f - off_t + j_t + 1.0
    E_K = jnp.where(
        jnp.abs(e_row.astype(f32) - jnp.broadcast_to(k_t, (A, TP))) < 0.5,
        1.0, 0.0).astype(f32)

    def expand(mat, E):
        return _nn(mat, E)

    fc_a = (0.5 * jnp.cos(d * (math.pi / Rca)) + 0.5)
    g = fc_a * jnp.where(d <= Rca, 1.0, 0.0) * ne

    dcj = expand(d, E_J)
    dck = expand(d, E_K)
    davg = expand(d, 0.5 * (E_J + E_K))
    dotp = (expand(dx, E_J) * expand(dx, E_K)
            + expand(dy, E_J) * expand(dy, E_K)
            + expand(dz, E_J) * expand(dz, E_K))

    cos_e = 0.95 * dotp / jnp.maximum(dcj * dck, 1e-10)
    sin_e = jnp.sqrt(jnp.maximum(1.0 - cos_e * cos_e, 0.0))

    base = 2.0 * expand(g, E_J) * expand(g, E_K)

    sjE = expand(spf, E_J)
    skE = expand(spf, E_K)
    mn = jnp.minimum(sjE, skE)
    mx = jnp.maximum(sjE, skE)
    clsE = mn * (2 * NS - mn - 1) * 0.5 + mx
    repP = jnp.where(_iota2((BM * NPAIRS, BM), 0) // NPAIRS
                     == _iota2((BM * NPAIRS, BM), 1), 1.0, 0.0).astype(f32)
    clsR = _nn(repP, clsE)
    W_cat = jnp.where(
        jnp.abs(clsR - (_iota2((BM * NPAIRS, TP), 0) % NPAIRS).astype(f32)) < 0.5,
        1.0, 0.0).astype(f32)

    hc = 0.5 * cos_e
    hs = 0.5 * sin_e
    f1s = []
    for z in range(NSHFZ):
        cz = math.cos(SHFZ[z])
        sz = math.sin(SHFZ[z])
        f1s.append(_pow_zeta((0.5 + hc * cz) + hs * sz))
    u_as = []
    for a in range(NSHFA):
        da = davg - SHFA[a]
        u_as.append(base * jnp.exp(-EtaA * da * da))

    GM = 4
    NG = BM // GM
    GR = GM * A
    GP = GM * NPAIRS
    mG = jnp.where(_iota2((GR, GP), 0) // A == _iota2((GR, GP), 1) // NPAIRS,
                   1.0, 0.0).astype(f32)
    qP = _iota2((GP, ANG_LEN), 1)
    pP = _iota2((GP, ANG_LEN), 0) % NPAIRS
    Pfs = [jnp.where(qP == pP * ANG_SUB + fidx, 1.0, 0.0).astype(f32)
           for fidx in range(ANG_SUB)]
    angs = [jnp.zeros((GR, ANG_LEN), dtype=f32) for _ in range(NG)]
    for a in range(NSHFA):
        for z in range(NSHFZ):
            at_f = u_as[a] * f1s[z]
            fidx = a * NSHFZ + z
            for grp in range(NG):
                at_g = at_f[grp * GR:(grp + 1) * GR]
                W_g = W_cat[grp * GP:(grp + 1) * GP]
                ang10 = _nt(at_g, W_g) * mG
                angs[grp] = angs[grp] + _nn(ang10, Pfs[fidx])
    ang = jnp.concatenate(angs, axis=0)

    out = jnp.concatenate([radial, ang], axis=-1)
    for b in range(BM):
        out_ref[b] = out[b * A:(b + 1) * A]


def kernel(species, coordinates):
    M, A = species.shape
    coords = coordinates.astype(f32)
    coords_t = jnp.swapaxes(coords, 1, 2)
    out = pl.pallas_call(
        _aev_kernel,
        grid=(M // BM,),
        in_specs=[
            pl.BlockSpec((BM, 1, A), lambda m: (m, 0, 0)),
            pl.BlockSpec((BM, A, 3), lambda m: (m, 0, 0)),
            pl.BlockSpec((BM, 3, A), lambda m: (m, 0, 0)),
        ],
        out_specs=pl.BlockSpec((BM, A, AEV_LEN), lambda m: (m, 0, 0)),
        out_shape=jax.ShapeDtypeStruct((M, A, AEV_LEN), jnp.float32),
        compiler_params=pltpu.CompilerParams(
            dimension_semantics=("parallel",),
        ),
    )(species.reshape(M, 1, A), coords, coords_t)
    return out


# speedup vs baseline: 1.6158x; 1.6158x over previous
import math

import jax
import jax.numpy as jnp
from jax.experimental import pallas as pl
from jax.experimental.pallas import tpu as pltpu

Rcr = 5.2
Rca = 3.5
NS = 4
EtaR = 16.0
EtaA = 8.0
NSHFR = 16
NSHFA = 4
NSHFZ = 8
ANG_SUB = NSHFA * NSHFZ
NPAIRS = NS * (NS + 1) // 2
RAD_LEN = NS * NSHFR
ANG_LEN = NPAIRS * ANG_SUB
AEV_LEN = RAD_LEN + ANG_LEN
BM = 32

SHFR = [0.9 + 0.26875 * f for f in range(NSHFR)]
SHFA = [0.9 + 0.65 * a for a in range(NSHFA)]
SHFZ = [(2 * z + 1) * math.pi / 16.0 for z in range(NSHFZ)]

f32 = jnp.float32
i32 = jnp.int32


def _pow_zeta(x):
    x = x * x
    x = x * x
    x = x * x
    x = x * x
    return x * x


def _iota2(shape, dim):
    return jax.lax.broadcasted_iota(i32, shape, dim)


def _nt(a, b):
    return jax.lax.dot_general(a, b, (((1,), (1,)), ((), ())),
                               preferred_element_type=f32)


def _nn(a, b):
    return jax.lax.dot_general(a, b, (((1,), (0,)), ((), ())),
                               preferred_element_type=f32)


def _aev_kernel(species_ref, coords_ref, coords_t_ref, out_ref):
    A = species_ref.shape[-1]
    JK = A * A
    R = BM * A

    dxs, dys, dzs = [], [], []
    for b in range(BM):
        xc = coords_ref[b]
        xr = coords_t_ref[b]
        dxs.append(jnp.broadcast_to(xc[:, 0:1], (A, A))
                   - jnp.broadcast_to(xr[0:1, :], (A, A)))
        dys.append(jnp.broadcast_to(xc[:, 1:2], (A, A))
                   - jnp.broadcast_to(xr[1:2, :], (A, A)))
        dzs.append(jnp.broadcast_to(xc[:, 2:3], (A, A))
                   - jnp.broadcast_to(xr[2:3, :], (A, A)))
    dx = jnp.concatenate(dxs, axis=0)
    dy = jnp.concatenate(dys, axis=0)
    dz = jnp.concatenate(dzs, axis=0)
    d = jnp.sqrt(dx * dx + dy * dy + dz * dz)

    ne = jnp.where(_iota2((R, A), 0) % A == _iota2((R, A), 1),
                   0.0, 1.0).astype(f32)

    spf = jnp.concatenate([species_ref[b] for b in range(BM)],
                          axis=0).astype(f32)

    fc_r = (0.5 * jnp.cos(d * (math.pi / Rcr)) + 0.5)
    fc_r = fc_r * jnp.where(d <= Rcr, 1.0, 0.0) * ne

    rep4 = jnp.where(_iota2((BM * NS, BM), 0) // NS == _iota2((BM * NS, BM), 1),
                     1.0, 0.0).astype(f32)
    spR4 = _nn(rep4, spf)
    S_cat = jnp.where(
        jnp.abs(spR4 - (_iota2((BM * NS, A), 0) % NS).astype(f32)) < 0.5,
        1.0, 0.0).astype(f32)

    m4 = jnp.where(_iota2((R, BM * NS), 0) // A == _iota2((R, BM * NS), 1) // NS,
                   1.0, 0.0).astype(f32)

    radial = jnp.zeros((R, RAD_LEN), dtype=f32)
    q4 = _iota2((BM * NS, RAD_LEN), 1)
    s4 = _iota2((BM * NS, RAD_LEN), 0) % NS
    for f in range(NSHFR):
        dd = d - SHFR[f]
        rt = 0.25 * jnp.exp(-EtaR * dd * dd) * fc_r
        rad = _nt(rt, S_cat) * m4
        Qf = jnp.where(q4 == s4 * NSHFR + f, 1.0, 0.0).astype(f32)
        radial = radial + _nn(rad, Qf)

    T = A * (A - 1) // 2
    TP = 512
    e_row = _iota2((A, TP), 0)
    e_col = _iota2((A, TP), 1)
    offj = e_row * (A - 1) - (e_row * (e_row - 1)) // 2
    E_J = jnp.where((e_col >= offj) & (e_col < offj + (A - 1 - e_row))
                    & (e_col < T), 1.0, 0.0).astype(f32)
    t_f = _iota2((1, TP), 1).astype(f32)
    disc = jnp.maximum((2 * A - 1) * (2 * A - 1) - 8.0 * t_f, 0.0)
    j_t = jnp.floor(((2 * A - 1) - jnp.sqrt(disc)) * 0.5)
    off_t = j_t * A - j_t * (j_t + 1.0) * 0.5
    k_t = t_f - off_t + j_t + 1.0
    E_K = jnp.where(
        jnp.abs(e_row.astype(f32) - jnp.broadcast_to(k_t, (A, TP))) < 0.5,
        1.0, 0.0).astype(f32)

    def expand(mat, E):
        return _nn(mat, E)

    fc_a = (0.5 * jnp.cos(d * (math.pi / Rca)) + 0.5)
    g = fc_a * jnp.where(d <= Rca, 1.0, 0.0) * ne

    dcj = expand(d, E_J)
    dck = expand(d, E_K)
    davg = expand(d, 0.5 * (E_J + E_K))
    dotp = (expand(dx, E_J) * expand(dx, E_K)
            + expand(dy, E_J) * expand(dy, E_K)
            + expand(dz, E_J) * expand(dz, E_K))

    cos_e = 0.95 * dotp / jnp.maximum(dcj * dck, 1e-10)
    sin_e = jnp.sqrt(jnp.maximum(1.0 - cos_e * cos_e, 0.0))

    base = 2.0 * expand(g, E_J) * expand(g, E_K)

    sjE = expand(spf, E_J)
    skE = expand(spf, E_K)
    mn = jnp.minimum(sjE, skE)
    mx = jnp.maximum(sjE, skE)
    clsE = mn * (2 * NS - mn - 1) * 0.5 + mx
    repP = jnp.where(_iota2((BM * NPAIRS, BM), 0) // NPAIRS
                     == _iota2((BM * NPAIRS, BM), 1), 1.0, 0.0).astype(f32)
    clsR = _nn(repP, clsE)
    W_cat = jnp.where(
        jnp.abs(clsR - (_iota2((BM * NPAIRS, TP), 0) % NPAIRS).astype(f32)) < 0.5,
        1.0, 0.0).astype(f32)

    hc = 0.5 * cos_e
    hs = 0.5 * sin_e
    f1s = []
    for z in range(NSHFZ):
        cz = math.cos(SHFZ[z])
        sz = math.sin(SHFZ[z])
        f1s.append(_pow_zeta((0.5 + hc * cz) + hs * sz))
    u_as = []
    for a in range(NSHFA):
        da = davg - SHFA[a]
        u_as.append(base * jnp.exp(-EtaA * da * da))

    GM = 16
    NG = BM // GM
    GR = GM * A
    GP = GM * NPAIRS
    mG = jnp.where(_iota2((GR, GP), 0) // A == _iota2((GR, GP), 1) // NPAIRS,
                   1.0, 0.0).astype(f32)
    qP = _iota2((GP, ANG_LEN), 1)
    pP = _iota2((GP, ANG_LEN), 0) % NPAIRS
    Pfs = [jnp.where(qP == pP * ANG_SUB + fidx, 1.0, 0.0).astype(f32)
           for fidx in range(ANG_SUB)]
    angs = [jnp.zeros((GR, ANG_LEN), dtype=f32) for _ in range(NG)]
    for a in range(NSHFA):
        for z in range(NSHFZ):
            at_f = u_as[a] * f1s[z]
            fidx = a * NSHFZ + z
            for grp in range(NG):
                at_g = at_f[grp * GR:(grp + 1) * GR]
                W_g = W_cat[grp * GP:(grp + 1) * GP]
                ang10 = _nt(at_g, W_g) * mG
                angs[grp] = angs[grp] + _nn(ang10, Pfs[fidx])
    ang = jnp.concatenate(angs, axis=0)

    out = jnp.concatenate([radial, ang], axis=-1)
    for b in range(BM):
        out_ref[b] = out[b * A:(b + 1) * A]


def kernel(species, coordinates):
    M, A = species.shape
    coords = coordinates.astype(f32)
    coords_t = jnp.swapaxes(coords, 1, 2)
    out = pl.pallas_call(
        _aev_kernel,
        grid=(M // BM,),
        in_specs=[
            pl.BlockSpec((BM, 1, A), lambda m: (m, 0, 0)),
            pl.BlockSpec((BM, A, 3), lambda m: (m, 0, 0)),
            pl.BlockSpec((BM, 3, A), lambda m: (m, 0, 0)),
        ],
        out_specs=pl.BlockSpec((BM, A, AEV_LEN), lambda m: (m, 0, 0)),
        out_shape=jax.ShapeDtypeStruct((M, A, AEV_LEN), jnp.float32),
        compiler_params=pltpu.CompilerParams(
            dimension_semantics=("parallel",),
        ),
    )(species.reshape(M, 1, A), coords, coords_t)
    return out
